# baseline (speedup 1.0000x reference)
import functools

import jax
import jax.numpy as jnp
from jax.experimental import pallas as pl
from jax.experimental.pallas import tpu as pltpu

_MARGIN = 1.0


def _mloss_block(sf_ref, tch_ref, cb_ref,
                 loss_ref, acc_ref, sat_ref, corr_ref, wrg_ref,
                 cbm2_ref, b2_ref, row_ref,
                 *, n_total, n_blocks):
    step = pl.program_id(0)

    @pl.when(step == 0)
    def _prep():
        cb = cb_ref[...]
        cbm2_ref[...] = -2.0 * cb
        b2_ref[...] = jnp.sum(cb * cb, axis=1, keepdims=True)
        row_ref[...] = jax.lax.broadcasted_iota(jnp.int32, row_ref.shape, 0)
        zero = jnp.zeros((1, 1), jnp.float32)
        loss_ref[...] = zero
        acc_ref[...] = zero
        sat_ref[...] = zero
        corr_ref[...] = zero
        wrg_ref[...] = zero

    sf = sf_ref[0]
    tch = tch_ref[0]

    e = jax.lax.dot_general(cbm2_ref[...], sf, (((1,), (0,)), ((), ())),
                            preferred_element_type=jnp.float32) + b2_ref[...]
    mask = row_ref[...] == tch

    correct_e = jnp.sum(jnp.where(mask, e, 0.0), axis=0, keepdims=True)
    wrong_e = jnp.min(jnp.where(mask, jnp.inf, e), axis=0, keepdims=True)

    a2 = jnp.sum(sf * sf, axis=0, keepdims=True)
    correct = jnp.sqrt(jnp.maximum(correct_e + a2, 0.0))
    wrong = jnp.sqrt(jnp.maximum(wrong_e + a2, 0.0))

    loss_ref[...] += jnp.sum(jnp.maximum(correct - wrong + _MARGIN, 0.0),
                             keepdims=True)
    acc_ref[...] += jnp.sum((correct_e < wrong_e).astype(jnp.float32),
                            keepdims=True)
    sat_ref[...] += jnp.sum((wrong - correct > _MARGIN).astype(jnp.float32),
                            keepdims=True)
    corr_ref[...] += jnp.sum(correct, keepdims=True)
    wrg_ref[...] += jnp.sum(wrong, keepdims=True)

    @pl.when(step == n_blocks - 1)
    def _finish():
        inv = 1.0 / n_total
        loss_ref[...] *= inv
        acc_ref[...] *= inv
        sat_ref[...] *= inv
        corr_ref[...] *= inv
        wrg_ref[...] *= inv


def kernel(student_features, teacher_codes, codebook):
    B, C, T_feat = student_features.shape
    if teacher_codes.ndim == 3:
        teacher_2d = teacher_codes[0]
    else:
        teacher_2d = jnp.squeeze(teacher_codes, axis=1)
    T = min(T_feat, teacher_2d.shape[1])
    K = codebook.shape[0]

    sf = student_features[:, :, :T]
    teacher = teacher_2d[:, :T].astype(jnp.int32)

    TBLK = 128
    t_per = T // TBLK
    n_blocks = B * t_per
    n_total = B * T
    teacher3 = teacher.reshape(n_blocks, 1, TBLK)

    body = functools.partial(_mloss_block, n_total=n_total, n_blocks=n_blocks)
    outs = pl.pallas_call(
        body,
        grid=(n_blocks,),
        in_specs=[
            pl.BlockSpec((1, C, TBLK), lambda rb: (rb // t_per, 0, rb % t_per)),
            pl.BlockSpec((1, 1, TBLK), lambda rb: (rb, 0, 0)),
            pl.BlockSpec((K, C), lambda rb: (0, 0)),
        ],
        out_specs=[pl.BlockSpec((1, 1), lambda rb: (0, 0))] * 5,
        out_shape=[jax.ShapeDtypeStruct((1, 1), jnp.float32)] * 5,
        scratch_shapes=[
            pltpu.VMEM((K, C), jnp.float32),
            pltpu.VMEM((K, 1), jnp.float32),
            pltpu.VMEM((K, TBLK), jnp.int32),
        ],
    )(sf, teacher3, codebook)

    loss, acc, sat, corr, wrg = (o[0, 0] for o in outs)
    return (loss, acc, sat, corr, wrg)


# speedup vs baseline: 1.5737x; 1.5737x over previous
import functools

import jax
import jax.numpy as jnp
from jax.experimental import pallas as pl

_MARGIN = 1.0


def _b2_prep(cb_ref, b2_ref):
    cb = cb_ref[...]
    b2_ref[...] = jnp.sum(cb * cb, axis=1, keepdims=True)


def _mloss_block(sf_ref, tch_ref, cb_ref, b2_ref, row_ref,
                 loss_ref, acc_ref, sat_ref, corr_ref, wrg_ref,
                 *, n_total, n_blocks):
    step = pl.program_id(0)

    @pl.when(step == 0)
    def _init():
        zero = jnp.zeros((1, 1), jnp.float32)
        loss_ref[...] = zero
        acc_ref[...] = zero
        sat_ref[...] = zero
        corr_ref[...] = zero
        wrg_ref[...] = zero

    sf = sf_ref[0]
    tch = tch_ref[0]

    e = jax.lax.dot_general(cb_ref[...], -2.0 * sf, (((1,), (0,)), ((), ())),
                            preferred_element_type=jnp.float32) + b2_ref[...]
    mask = row_ref[...] == tch

    correct_e = jnp.sum(jnp.where(mask, e, 0.0), axis=0, keepdims=True)
    wrong_e = jnp.min(jnp.where(mask, jnp.inf, e), axis=0, keepdims=True)

    a2 = jnp.sum(sf * sf, axis=0, keepdims=True)
    correct = jnp.sqrt(jnp.maximum(correct_e + a2, 0.0))
    wrong = jnp.sqrt(jnp.maximum(wrong_e + a2, 0.0))

    loss_ref[...] += jnp.sum(jnp.maximum(correct - wrong + _MARGIN, 0.0),
                             keepdims=True)
    acc_ref[...] += jnp.sum((correct_e < wrong_e).astype(jnp.float32),
                            keepdims=True)
    sat_ref[...] += jnp.sum((wrong - correct > _MARGIN).astype(jnp.float32),
                            keepdims=True)
    corr_ref[...] += jnp.sum(correct, keepdims=True)
    wrg_ref[...] += jnp.sum(wrong, keepdims=True)

    @pl.when(step == n_blocks - 1)
    def _finish():
        inv = 1.0 / n_total
        loss_ref[...] *= inv
        acc_ref[...] *= inv
        sat_ref[...] *= inv
        corr_ref[...] *= inv
        wrg_ref[...] *= inv


def kernel(student_features, teacher_codes, codebook):
    B, C, T_feat = student_features.shape
    if teacher_codes.ndim == 3:
        teacher_2d = teacher_codes[0]
    else:
        teacher_2d = jnp.squeeze(teacher_codes, axis=1)
    T = min(T_feat, teacher_2d.shape[1])
    K = codebook.shape[0]

    sf = student_features[:, :, :T]
    teacher = teacher_2d[:, :T].astype(jnp.int32)

    TBLK = 256
    t_per = T // TBLK
    n_blocks = B * t_per
    n_total = B * T
    teacher3 = teacher.reshape(n_blocks, 1, TBLK)
    row = jax.lax.broadcasted_iota(jnp.int32, (K, TBLK), 0)

    b2 = pl.pallas_call(
        _b2_prep,
        out_shape=jax.ShapeDtypeStruct((K, 1), jnp.float32),
    )(codebook)

    body = functools.partial(_mloss_block, n_total=n_total, n_blocks=n_blocks)
    outs = pl.pallas_call(
        body,
        grid=(n_blocks,),
        in_specs=[
            pl.BlockSpec((1, C, TBLK), lambda rb: (rb // t_per, 0, rb % t_per)),
            pl.BlockSpec((1, 1, TBLK), lambda rb: (rb, 0, 0)),
            pl.BlockSpec((K, C), lambda rb: (0, 0)),
            pl.BlockSpec((K, 1), lambda rb: (0, 0)),
            pl.BlockSpec((K, TBLK), lambda rb: (0, 0)),
        ],
        out_specs=[pl.BlockSpec((1, 1), lambda rb: (0, 0))] * 5,
        out_shape=[jax.ShapeDtypeStruct((1, 1), jnp.float32)] * 5,
    )(sf, teacher3, codebook, b2, row)

    loss, acc, sat, corr, wrg = (o[0, 0] for o in outs)
    return (loss, acc, sat, corr, wrg)


# speedup vs baseline: 1.8332x; 1.1649x over previous
import functools

import jax
import jax.numpy as jnp
from jax.experimental import pallas as pl
from jax.experimental.pallas import tpu as pltpu

_MARGIN = 1.0


def _mloss_block(sf_ref, tch_ref, cb_ref,
                 loss_ref, acc_ref, sat_ref, corr_ref, wrg_ref,
                 b2_ref, row_ref,
                 *, n_total, n_blocks):
    step = pl.program_id(0)

    @pl.when(step == 0)
    def _prep():
        cb = cb_ref[...]
        b2_ref[...] = jnp.sum(cb * cb, axis=1, keepdims=True)
        row_ref[...] = jax.lax.broadcasted_iota(jnp.int32, row_ref.shape, 0)
        zero = jnp.zeros((1, 1), jnp.float32)
        loss_ref[...] = zero
        acc_ref[...] = zero
        sat_ref[...] = zero
        corr_ref[...] = zero
        wrg_ref[...] = zero

    sf = sf_ref[0]
    tch = tch_ref[0]

    e = jax.lax.dot_general(cb_ref[...], -2.0 * sf, (((1,), (0,)), ((), ())),
                            preferred_element_type=jnp.float32) + b2_ref[...]
    mask = row_ref[...] == tch

    correct_e = jnp.sum(jnp.where(mask, e, 0.0), axis=0, keepdims=True)
    wrong_e = jnp.min(jnp.where(mask, jnp.inf, e), axis=0, keepdims=True)

    a2 = jnp.sum(sf * sf, axis=0, keepdims=True)
    correct = jnp.sqrt(jnp.maximum(correct_e + a2, 0.0))
    wrong = jnp.sqrt(jnp.maximum(wrong_e + a2, 0.0))

    loss_ref[...] += jnp.sum(jnp.maximum(correct - wrong + _MARGIN, 0.0),
                             keepdims=True)
    acc_ref[...] += jnp.sum((correct_e < wrong_e).astype(jnp.float32),
                            keepdims=True)
    sat_ref[...] += jnp.sum((wrong - correct > _MARGIN).astype(jnp.float32),
                            keepdims=True)
    corr_ref[...] += jnp.sum(correct, keepdims=True)
    wrg_ref[...] += jnp.sum(wrong, keepdims=True)

    @pl.when(step == n_blocks - 1)
    def _finish():
        inv = 1.0 / n_total
        loss_ref[...] *= inv
        acc_ref[...] *= inv
        sat_ref[...] *= inv
        corr_ref[...] *= inv
        wrg_ref[...] *= inv


def kernel(student_features, teacher_codes, codebook):
    B, C, T_feat = student_features.shape
    if teacher_codes.ndim == 3:
        teacher_2d = teacher_codes[0]
    else:
        teacher_2d = jnp.squeeze(teacher_codes, axis=1)
    T = min(T_feat, teacher_2d.shape[1])
    K = codebook.shape[0]

    sf = student_features[:, :, :T]
    teacher = teacher_2d[:, :T].astype(jnp.int32)

    TBLK = 256
    t_per = T // TBLK
    n_blocks = B * t_per
    n_total = B * T
    teacher3 = teacher.reshape(n_blocks, 1, TBLK)

    body = functools.partial(_mloss_block, n_total=n_total, n_blocks=n_blocks)
    outs = pl.pallas_call(
        body,
        grid=(n_blocks,),
        in_specs=[
            pl.BlockSpec((1, C, TBLK), lambda rb: (rb // t_per, 0, rb % t_per)),
            pl.BlockSpec((1, 1, TBLK), lambda rb: (rb, 0, 0)),
            pl.BlockSpec((K, C), lambda rb: (0, 0)),
        ],
        out_specs=[pl.BlockSpec((1, 1), lambda rb: (0, 0))] * 5,
        out_shape=[jax.ShapeDtypeStruct((1, 1), jnp.float32)] * 5,
        scratch_shapes=[
            pltpu.VMEM((K, 1), jnp.float32),
            pltpu.VMEM((K, TBLK), jnp.int32),
        ],
    )(sf, teacher3, codebook)

    loss, acc, sat, corr, wrg = (o[0, 0] for o in outs)
    return (loss, acc, sat, corr, wrg)
